# Initial kernel scaffold; baseline (speedup 1.0000x reference)
import jax
import jax.numpy as jnp
from jax.experimental import pallas as pl


def kernel(x, lin_x, params, edge_index, batch_vec, lin_edge_index):
    raise NotImplementedError("write your pallas kernel here")


# speedup vs baseline: 8.5039x; 8.5039x over previous
import functools
import math

import jax
import jax.numpy as jnp
import numpy as np
from jax.experimental import pallas as pl

_D = 128
_HEADS = 4
_DH = _D // _HEADS
_LN_EPS = 1e-5


def _mm_body(a_ref, w_ref, b_ref, o_ref, *, act):
    y = jnp.dot(a_ref[...], w_ref[...], preferred_element_type=jnp.float32)
    y = y + b_ref[...]
    if act == "relu":
        y = jnp.maximum(y, 0.0)
    o_ref[...] = y


def _mm(a, w, b=None, act=None, block=512):
    M, K = a.shape
    N = w.shape[1]
    Kp = ((K + 127) // 128) * 128
    Mp = ((M + block - 1) // block) * block
    if Kp != K:
        a = jnp.pad(a, ((0, 0), (0, Kp - K)))
        w = jnp.pad(w, ((0, Kp - K), (0, 0)))
    if Mp != M:
        a = jnp.pad(a, ((0, Mp - M), (0, 0)))
    if b is None:
        b = jnp.zeros((N,), jnp.float32)
    b2 = b.reshape(1, N)
    out = pl.pallas_call(
        functools.partial(_mm_body, act=act),
        grid=(Mp // block,),
        in_specs=[
            pl.BlockSpec((block, Kp), lambda i: (i, 0)),
            pl.BlockSpec((Kp, N), lambda i: (0, 0)),
            pl.BlockSpec((1, N), lambda i: (0, 0)),
        ],
        out_specs=pl.BlockSpec((block, N), lambda i: (i, 0)),
        out_shape=jax.ShapeDtypeStruct((Mp, N), jnp.float32),
    )(a, w, b2)
    return out[:M]


def _ln(x, g, b):
    m = jnp.mean(x, axis=-1, keepdims=True)
    v = jnp.var(x, axis=-1, keepdims=True)
    return (x - m) / jnp.sqrt(v + _LN_EPS) * g + b


def _edge_pass(h, ea_tab, src, dst, aidx, a_vec, n_seg):
    hs = h[src]
    hd = h[dst]
    ea = ea_tab if aidx is None else ea_tab[aidx]
    t = jax.nn.leaky_relu(hs + hd + ea, 0.2).reshape(-1, _HEADS, _DH)
    logits = jnp.sum(t * a_vec[None], axis=-1)
    e = jnp.exp(logits)
    msg = (hs + ea).reshape(-1, _HEADS, _DH)
    u = jax.ops.segment_sum((e[..., None] * msg).reshape(-1, _D), dst,
                            num_segments=n_seg)
    s = jax.ops.segment_sum(e, dst, num_segments=n_seg)
    return u, s


def _finish_body(x_ref, u_ref, s_ref, w_ref, b_ref, g_ref, bb_ref, o_ref):
    s = s_ref[...]
    alpha = 1.0 / (s + 1e-16)
    u = u_ref[...].reshape(s.shape[0], _HEADS, _DH)
    agg = (u * alpha[..., None]).reshape(s.shape[0], _D)
    y = x_ref[...] + jnp.dot(agg, w_ref[...],
                             preferred_element_type=jnp.float32) + b_ref[...]
    m = jnp.mean(y, axis=-1, keepdims=True)
    v = jnp.mean((y - m) * (y - m), axis=-1, keepdims=True)
    o_ref[...] = (y - m) / jnp.sqrt(v + _LN_EPS) * g_ref[...] + bb_ref[...]


def _finish(x, u, s, p, block=512):
    M = x.shape[0]
    Mp = ((M + block - 1) // block) * block
    if Mp != M:
        x = jnp.pad(x, ((0, Mp - M), (0, 0)))
        u = jnp.pad(u, ((0, Mp - M), (0, 0)))
        s = jnp.pad(s, ((0, Mp - M), (0, 0)), constant_values=1.0)
    out = pl.pallas_call(
        _finish_body,
        grid=(Mp // block,),
        in_specs=[
            pl.BlockSpec((block, _D), lambda i: (i, 0)),
            pl.BlockSpec((block, _D), lambda i: (i, 0)),
            pl.BlockSpec((block, _HEADS), lambda i: (i, 0)),
            pl.BlockSpec((_D, _D), lambda i: (0, 0)),
            pl.BlockSpec((1, _D), lambda i: (0, 0)),
            pl.BlockSpec((1, _D), lambda i: (0, 0)),
            pl.BlockSpec((1, _D), lambda i: (0, 0)),
        ],
        out_specs=pl.BlockSpec((block, _D), lambda i: (i, 0)),
        out_shape=jax.ShapeDtypeStruct((Mp, _D), jnp.float32),
    )(x, u, s, p["o"]["W"], p["o"]["b"].reshape(1, _D),
      p["ln_g"].reshape(1, _D), p["ln_b"].reshape(1, _D))
    return out[:M]


def _pos_enc(L, d):
    pos = np.arange(L)[:, None].astype(np.float32)
    i = np.arange(d)[None, :].astype(np.float32)
    ang = pos / np.power(10000.0, (2.0 * np.floor(i / 2.0)) / d)
    pe = np.zeros((L, d), np.float32)
    pe[:, 0::2] = np.sin(ang[:, 0::2])
    pe[:, 1::2] = np.cos(ang[:, 1::2])
    return jnp.asarray(pe)


def _attn_body(x_ref, mask_ref, wq_ref, bq_ref, wk_ref, bk_ref, wv_ref,
               bv_ref, wo_ref, bo_ref, g_ref, bb_ref, o_ref):
    x = x_ref[0]
    q = jnp.dot(x, wq_ref[...], preferred_element_type=jnp.float32) + bq_ref[...]
    k = jnp.dot(x, wk_ref[...], preferred_element_type=jnp.float32) + bk_ref[...]
    v = jnp.dot(x, wv_ref[...], preferred_element_type=jnp.float32) + bv_ref[...]
    mask = mask_ref[0]
    outs = []
    scale = 1.0 / math.sqrt(_DH)
    for h in range(_HEADS):
        qh = q[:, h * _DH:(h + 1) * _DH]
        kh = k[:, h * _DH:(h + 1) * _DH]
        vh = v[:, h * _DH:(h + 1) * _DH]
        sc = jax.lax.dot_general(qh, kh, (((1,), (1,)), ((), ())),
                                 preferred_element_type=jnp.float32)
        sc = sc * scale + mask
        sc = sc - jnp.max(sc, axis=-1, keepdims=True)
        p = jnp.exp(sc)
        p = p / jnp.sum(p, axis=-1, keepdims=True)
        outs.append(jnp.dot(p, vh, preferred_element_type=jnp.float32))
    o = jnp.concatenate(outs, axis=-1)
    y = x + jnp.dot(o, wo_ref[...], preferred_element_type=jnp.float32) + bo_ref[...]
    m = jnp.mean(y, axis=-1, keepdims=True)
    var = jnp.mean((y - m) * (y - m), axis=-1, keepdims=True)
    o_ref[0] = (y - m) / jnp.sqrt(var + _LN_EPS) * g_ref[...] + bb_ref[...]


def _attention(x, att_mask, p):
    G, L, _ = x.shape
    out = pl.pallas_call(
        _attn_body,
        grid=(G,),
        in_specs=[
            pl.BlockSpec((1, L, _D), lambda i: (i, 0, 0)),
            pl.BlockSpec((1, 1, L), lambda i: (i, 0, 0)),
            pl.BlockSpec((_D, _D), lambda i: (0, 0)),
            pl.BlockSpec((1, _D), lambda i: (0, 0)),
            pl.BlockSpec((_D, _D), lambda i: (0, 0)),
            pl.BlockSpec((1, _D), lambda i: (0, 0)),
            pl.BlockSpec((_D, _D), lambda i: (0, 0)),
            pl.BlockSpec((1, _D), lambda i: (0, 0)),
            pl.BlockSpec((_D, _D), lambda i: (0, 0)),
            pl.BlockSpec((1, _D), lambda i: (0, 0)),
            pl.BlockSpec((1, _D), lambda i: (0, 0)),
            pl.BlockSpec((1, _D), lambda i: (0, 0)),
        ],
        out_specs=pl.BlockSpec((1, L, _D), lambda i: (i, 0, 0)),
        out_shape=jax.ShapeDtypeStruct((G, L, _D), jnp.float32),
    )(x, att_mask,
      p["q"]["W"], p["q"]["b"].reshape(1, _D),
      p["k"]["W"], p["k"]["b"].reshape(1, _D),
      p["v"]["W"], p["v"]["b"].reshape(1, _D),
      p["o"]["W"], p["o"]["b"].reshape(1, _D),
      p["ln_g"].reshape(1, _D), p["ln_b"].reshape(1, _D))
    return out


def kernel(x, lin_x, params, edge_index, batch_vec, lin_edge_index):
    N = x.shape[0]
    E = edge_index.shape[1]
    G = 256
    L = 256

    node_x = _mm(x, params["atom_init"]["W"], params["atom_init"]["b"])
    edge_x = _mm(lin_x, params["bond_init"]["W"], params["bond_init"]["b"])

    src, dst = edge_index[0], edge_index[1]
    lsrc, ldst = lin_edge_index[0], lin_edge_index[1]
    gidx = src[lsrc]

    for i in range(3):
        pa = params["atom_blocks"][i]
        pb = params["bond_blocks"][i]
        node_old = node_x
        edge_old = edge_x
        h_a = _mm(node_old, pa["h"]["W"], pa["h"]["b"])
        ea_a = _mm(edge_old, pa["We"])
        u_a, s_a = _edge_pass(h_a, ea_a, src, dst, None, pa["a"], N)
        node_x = _finish(node_old, u_a, s_a, pa)
        h_b = _mm(edge_old, pb["h"]["W"], pb["h"]["b"])
        nw = _mm(node_old, pb["We"])
        u_b, s_b = _edge_pass(h_b, nw, lsrc, ldst, gidx, pb["a"], E)
        edge_x = _finish(edge_old, u_b, s_b, pb)

    e2n = jax.ops.segment_sum(edge_x, dst, num_segments=N)
    edge_pooled = _mm(e2n, params["pool_edge"]["W"], params["pool_edge"]["b"],
                      act="relu")
    graph_z = _mm(jnp.concatenate([node_x, edge_pooled], axis=-1),
                  params["merge1"]["W"], params["merge1"]["b"], act="relu")

    counts = jnp.bincount(batch_vec, length=G)
    starts = jnp.concatenate([jnp.zeros((1,), counts.dtype),
                              jnp.cumsum(counts)[:-1]])
    pos = jnp.arange(N, dtype=batch_vec.dtype) - starts[batch_vec].astype(batch_vec.dtype)
    dense = jnp.zeros((G, L, _D), jnp.float32).at[batch_vec, pos].set(graph_z)
    mask = jnp.zeros((G, L), bool).at[batch_vec, pos].set(True)
    att_mask = jnp.where(mask, 0.0, -1e9).astype(jnp.float32)[:, None, :]

    seq = _ln(dense + _pos_enc(L, _D)[None], params["norm_g"], params["norm_b"])
    seq_z = _attention(seq, att_mask, params["attn"])
    seq_flat = seq_z[batch_vec, pos]

    z = _mm(jnp.concatenate([graph_z, seq_flat], axis=-1),
            params["merge2"]["W"], params["merge2"]["b"], act="relu")

    cnt = jnp.maximum(counts, 1).astype(jnp.float32)[:, None]

    def _pool(feat, pp, ff):
        g = jax.ops.segment_sum(feat, batch_vec, num_segments=G) / cnt
        hmid = _mm(g, pp["W"], pp["b"], act="relu", block=256)
        return _mm(hmid, ff["W"], ff["b"], block=256)

    z_out = _pool(z, params["pool_graph"], params["ffnn"])
    z1 = _pool(node_x, params["pool_z1"], params["ffnn_z1"])
    z2 = _pool(edge_pooled, params["pool_z2"], params["ffnn_z2"])
    return (z_out, z1, z2)


# speedup vs baseline: 8.7174x; 1.0251x over previous
import functools
import math

import jax
import jax.numpy as jnp
import numpy as np
from jax import lax
from jax.experimental import pallas as pl
from jax.experimental.pallas import tpu as pltpu
from jax.experimental.pallas import tpu_sc as plsc

_D = 128
_HEADS = 4
_DH = _D // _HEADS
_LN_EPS = 1e-5
_B = 64

_GDN = lax.GatherDimensionNumbers(offset_dims=(), collapsed_slice_dims=(0,),
                                  start_index_map=(0,))


def _shuf(v, idx):
    return lax.gather(v, idx[:, None], _GDN, (1,),
                      mode=lax.GatherScatterMode.PROMISE_IN_BOUNDS)


def _hsum(v, lanes):
    for d in (1, 2, 4, 8):
        v = v + _shuf(v, lanes ^ d)
    return v


def _edge_pass_body(h_hbm, ea_hbm, src_hbm, dst_hbm, aidx_hbm, cs_hbm, a_hbm,
                    u_hbm, s_hbm,
                    cs_v, a_v, src_v, dst_v, aidx_v, drel_v,
                    hs_v, hd_v, ea_v, msg_v, s_v, zt_v,
                    acc_u, acc_s, sem1, sem2, sem3,
                    *, n_chunks, chunk):
    cid = lax.axis_index("c")
    sid = lax.axis_index("s")
    rows = chunk // 16
    pltpu.sync_copy(cs_hbm, cs_v)
    pltpu.sync_copy(a_hbm, a_v)
    a_regs = [a_v[pl.ds(16 * k, 16)] for k in range(8)]
    lanes = lax.iota(jnp.int32, 16)

    zf = jnp.zeros((16,), jnp.float32)

    def _zero_rows(j, _):
        for k in range(8):
            zt_v[j, pl.ds(16 * k, 16)] = zf
        return 0

    lax.fori_loop(0, _B, _zero_rows, 0)

    def _block(b, carry):
        seg_base, my_lo, my_hi, base8 = carry
        base = base8 + b * _B
        pltpu.sync_copy(src_hbm.at[pl.ds(base, _B)], src_v)
        pltpu.sync_copy(dst_hbm.at[pl.ds(base, _B)], dst_v)
        pltpu.sync_copy(aidx_hbm.at[pl.ds(base, _B)], aidx_v)
        for k in range(_B // 16):
            d = dst_v[pl.ds(16 * k, 16)] - seg_base
            drel_v[pl.ds(16 * k, 16)] = jnp.clip(d, 0, chunk - 1)
        c1 = pltpu.async_copy(h_hbm.at[src_v], hs_v, sem1)
        c2 = pltpu.async_copy(h_hbm.at[dst_v], hd_v, sem2)
        c3 = pltpu.async_copy(ea_hbm.at[aidx_v], ea_v, sem3)
        c1.wait()
        c2.wait()
        c3.wait()

        def _edge(j, _):
            eid = base + j
            flag = jnp.where((eid >= my_lo) & (eid < my_hi), 1.0, 0.0)
            flag_v = lax.broadcast(flag, (16,))
            hs = [hs_v[j, pl.ds(16 * k, 16)] for k in range(8)]
            hd = [hd_v[j, pl.ds(16 * k, 16)] for k in range(8)]
            ea = [ea_v[j, pl.ds(16 * k, 16)] for k in range(8)]
            es = []
            for hh in range(4):
                k0, k1 = 2 * hh, 2 * hh + 1
                t0 = hs[k0] + hd[k0] + ea[k0]
                t1 = hs[k1] + hd[k1] + ea[k1]
                t0 = jnp.maximum(t0, 0.2 * t0)
                t1 = jnp.maximum(t1, 0.2 * t1)
                logit = _hsum(t0 * a_regs[k0] + t1 * a_regs[k1], lanes)
                es.append(jnp.exp(logit) * flag_v)
            for k in range(8):
                msg_v[j, pl.ds(16 * k, 16)] = (hs[k] + ea[k]) * es[k // 2]
                s_v[j, pl.ds(16 * k, 16)] = es[min(k, 3)]
            return 0

        lax.fori_loop(0, _B, _edge, 0)
        pltpu.sync_copy(msg_v, acc_u.at[drel_v], add=True)
        pltpu.sync_copy(s_v, acc_s.at[drel_v], add=True)
        return carry

    def _chunk(i, _):
        c = cid + 2 * i
        seg_base = c * chunk
        row0 = sid * rows
        for q in range(rows // _B):
            pltpu.sync_copy(zt_v, acc_u.at[pl.ds(row0 + q * _B, _B)])
            pltpu.sync_copy(zt_v, acc_s.at[pl.ds(row0 + q * _B, _B)])
        plsc.subcore_barrier()
        e_lo = cs_v[pl.ds(c, 16)][0]
        e_hi = cs_v[pl.ds(c + 1, 16)][0]
        ln = e_hi - e_lo
        per = (ln + 15) // 16
        my_lo = jnp.minimum(e_lo + sid * per, e_hi)
        my_hi = jnp.minimum(my_lo + per, e_hi)
        base8 = (my_lo // 8) * 8
        nblk = (my_hi - base8 + _B - 1) // _B
        lax.fori_loop(0, nblk, _block, (seg_base, my_lo, my_hi, base8))
        plsc.subcore_barrier()
        pltpu.sync_copy(acc_u.at[pl.ds(row0, rows)],
                        u_hbm.at[pl.ds(seg_base + row0, rows)])
        pltpu.sync_copy(acc_s.at[pl.ds(row0, rows)],
                        s_hbm.at[pl.ds(seg_base + row0, rows)])
        return 0

    lax.fori_loop(0, n_chunks // 2, _chunk, 0)


def _edge_pass_sc(h, ea_tab, src_s, dst_s, aidx_s, chunk_starts, a_flat,
                  n_chunks, chunk):
    assert n_chunks % 2 == 0 and chunk % 1024 == 0
    nseg_pad = n_chunks * chunk
    cs_len = chunk_starts.shape[0]
    mesh = plsc.VectorSubcoreMesh(core_axis_name="c", subcore_axis_name="s")
    body = functools.partial(_edge_pass_body, n_chunks=n_chunks, chunk=chunk)
    fn = pl.kernel(
        body,
        out_type=[jax.ShapeDtypeStruct((nseg_pad, _D), jnp.float32),
                  jax.ShapeDtypeStruct((nseg_pad, _D), jnp.float32)],
        mesh=mesh,
        scratch_types=[
            pltpu.VMEM((cs_len,), jnp.int32),
            pltpu.VMEM((_D,), jnp.float32),
            pltpu.VMEM((_B,), jnp.int32),
            pltpu.VMEM((_B,), jnp.int32),
            pltpu.VMEM((_B,), jnp.int32),
            pltpu.VMEM((_B,), jnp.int32),
            pltpu.VMEM((_B, _D), jnp.float32),
            pltpu.VMEM((_B, _D), jnp.float32),
            pltpu.VMEM((_B, _D), jnp.float32),
            pltpu.VMEM((_B, _D), jnp.float32),
            pltpu.VMEM((_B, _D), jnp.float32),
            pltpu.VMEM((_B, _D), jnp.float32),
            pltpu.VMEM_SHARED((chunk, _D), jnp.float32),
            pltpu.VMEM_SHARED((chunk, _D), jnp.float32),
            pltpu.SemaphoreType.DMA,
            pltpu.SemaphoreType.DMA,
            pltpu.SemaphoreType.DMA,
        ],
    )
    return fn(h, ea_tab, src_s, dst_s, aidx_s, chunk_starts, a_flat)


def _mm_body(a_ref, w_ref, b_ref, o_ref, *, act):
    y = jnp.dot(a_ref[...], w_ref[...], preferred_element_type=jnp.float32)
    y = y + b_ref[...]
    if act == "relu":
        y = jnp.maximum(y, 0.0)
    o_ref[...] = y


def _mm(a, w, b=None, act=None, block=512):
    M, K = a.shape
    N = w.shape[1]
    Kp = ((K + 127) // 128) * 128
    Mp = ((M + block - 1) // block) * block
    if Kp != K:
        a = jnp.pad(a, ((0, 0), (0, Kp - K)))
        w = jnp.pad(w, ((0, Kp - K), (0, 0)))
    if Mp != M:
        a = jnp.pad(a, ((0, Mp - M), (0, 0)))
    if b is None:
        b = jnp.zeros((N,), jnp.float32)
    b2 = b.reshape(1, N)
    out = pl.pallas_call(
        functools.partial(_mm_body, act=act),
        grid=(Mp // block,),
        in_specs=[
            pl.BlockSpec((block, Kp), lambda i: (i, 0)),
            pl.BlockSpec((Kp, N), lambda i: (0, 0)),
            pl.BlockSpec((1, N), lambda i: (0, 0)),
        ],
        out_specs=pl.BlockSpec((block, N), lambda i: (i, 0)),
        out_shape=jax.ShapeDtypeStruct((Mp, N), jnp.float32),
    )(a, w, b2)
    return out[:M]


def _s_lanes(s_full, n):
    return s_full[:n].reshape(n, 8, 16)[:, :_HEADS, 0]


def _ln(x, g, b):
    m = jnp.mean(x, axis=-1, keepdims=True)
    v = jnp.var(x, axis=-1, keepdims=True)
    return (x - m) / jnp.sqrt(v + _LN_EPS) * g + b


def _sorted_edges(dst, n_chunks, chunk, pad=256):
    perm = jnp.argsort(dst)
    dst_s = dst[perm]
    bounds = jnp.arange(n_chunks + 1, dtype=jnp.int32) * chunk
    cs = jnp.searchsorted(dst_s, bounds).astype(jnp.int32)
    cs_len = ((n_chunks + 16) + 7) // 8 * 8
    cs = jnp.pad(cs, (0, cs_len - cs.shape[0]),
                 constant_values=dst.shape[0])
    dst_p = jnp.pad(dst_s, (0, pad))
    perm_p = jnp.pad(perm, (0, pad)).astype(jnp.int32)
    return perm, perm_p, dst_p, cs


def _finish_body(x_ref, u_ref, s_ref, w_ref, b_ref, g_ref, bb_ref, o_ref):
    s = s_ref[...]
    alpha = 1.0 / (s + 1e-16)
    u = u_ref[...].reshape(s.shape[0], _HEADS, _DH)
    agg = (u * alpha[..., None]).reshape(s.shape[0], _D)
    y = x_ref[...] + jnp.dot(agg, w_ref[...],
                             preferred_element_type=jnp.float32) + b_ref[...]
    m = jnp.mean(y, axis=-1, keepdims=True)
    v = jnp.mean((y - m) * (y - m), axis=-1, keepdims=True)
    o_ref[...] = (y - m) / jnp.sqrt(v + _LN_EPS) * g_ref[...] + bb_ref[...]


def _finish(x, u, s, p, block=512):
    M = x.shape[0]
    Mp = ((M + block - 1) // block) * block
    if Mp != M:
        x = jnp.pad(x, ((0, Mp - M), (0, 0)))
        u = jnp.pad(u, ((0, Mp - M), (0, 0)))
        s = jnp.pad(s, ((0, Mp - M), (0, 0)), constant_values=1.0)
    out = pl.pallas_call(
        _finish_body,
        grid=(Mp // block,),
        in_specs=[
            pl.BlockSpec((block, _D), lambda i: (i, 0)),
            pl.BlockSpec((block, _D), lambda i: (i, 0)),
            pl.BlockSpec((block, _HEADS), lambda i: (i, 0)),
            pl.BlockSpec((_D, _D), lambda i: (0, 0)),
            pl.BlockSpec((1, _D), lambda i: (0, 0)),
            pl.BlockSpec((1, _D), lambda i: (0, 0)),
            pl.BlockSpec((1, _D), lambda i: (0, 0)),
        ],
        out_specs=pl.BlockSpec((block, _D), lambda i: (i, 0)),
        out_shape=jax.ShapeDtypeStruct((Mp, _D), jnp.float32),
    )(x, u, s, p["o"]["W"], p["o"]["b"].reshape(1, _D),
      p["ln_g"].reshape(1, _D), p["ln_b"].reshape(1, _D))
    return out[:M]


def _pos_enc(L, d):
    pos = np.arange(L)[:, None].astype(np.float32)
    i = np.arange(d)[None, :].astype(np.float32)
    ang = pos / np.power(10000.0, (2.0 * np.floor(i / 2.0)) / d)
    pe = np.zeros((L, d), np.float32)
    pe[:, 0::2] = np.sin(ang[:, 0::2])
    pe[:, 1::2] = np.cos(ang[:, 1::2])
    return jnp.asarray(pe)


def _attn_body(x_ref, mask_ref, wq_ref, bq_ref, wk_ref, bk_ref, wv_ref,
               bv_ref, wo_ref, bo_ref, g_ref, bb_ref, o_ref):
    x = x_ref[0]
    q = jnp.dot(x, wq_ref[...], preferred_element_type=jnp.float32) + bq_ref[...]
    k = jnp.dot(x, wk_ref[...], preferred_element_type=jnp.float32) + bk_ref[...]
    v = jnp.dot(x, wv_ref[...], preferred_element_type=jnp.float32) + bv_ref[...]
    mask = mask_ref[0]
    outs = []
    scale = 1.0 / math.sqrt(_DH)
    for h in range(_HEADS):
        qh = q[:, h * _DH:(h + 1) * _DH]
        kh = k[:, h * _DH:(h + 1) * _DH]
        vh = v[:, h * _DH:(h + 1) * _DH]
        sc = jax.lax.dot_general(qh, kh, (((1,), (1,)), ((), ())),
                                 preferred_element_type=jnp.float32)
        sc = sc * scale + mask
        sc = sc - jnp.max(sc, axis=-1, keepdims=True)
        p = jnp.exp(sc)
        p = p / jnp.sum(p, axis=-1, keepdims=True)
        outs.append(jnp.dot(p, vh, preferred_element_type=jnp.float32))
    o = jnp.concatenate(outs, axis=-1)
    y = x + jnp.dot(o, wo_ref[...], preferred_element_type=jnp.float32) + bo_ref[...]
    m = jnp.mean(y, axis=-1, keepdims=True)
    var = jnp.mean((y - m) * (y - m), axis=-1, keepdims=True)
    o_ref[0] = (y - m) / jnp.sqrt(var + _LN_EPS) * g_ref[...] + bb_ref[...]


def _attention(x, att_mask, p):
    G, L, _ = x.shape
    out = pl.pallas_call(
        _attn_body,
        grid=(G,),
        in_specs=[
            pl.BlockSpec((1, L, _D), lambda i: (i, 0, 0)),
            pl.BlockSpec((1, 1, L), lambda i: (i, 0, 0)),
            pl.BlockSpec((_D, _D), lambda i: (0, 0)),
            pl.BlockSpec((1, _D), lambda i: (0, 0)),
            pl.BlockSpec((_D, _D), lambda i: (0, 0)),
            pl.BlockSpec((1, _D), lambda i: (0, 0)),
            pl.BlockSpec((_D, _D), lambda i: (0, 0)),
            pl.BlockSpec((1, _D), lambda i: (0, 0)),
            pl.BlockSpec((_D, _D), lambda i: (0, 0)),
            pl.BlockSpec((1, _D), lambda i: (0, 0)),
            pl.BlockSpec((1, _D), lambda i: (0, 0)),
            pl.BlockSpec((1, _D), lambda i: (0, 0)),
        ],
        out_specs=pl.BlockSpec((1, L, _D), lambda i: (i, 0, 0)),
        out_shape=jax.ShapeDtypeStruct((G, L, _D), jnp.float32),
    )(x, att_mask,
      p["q"]["W"], p["q"]["b"].reshape(1, _D),
      p["k"]["W"], p["k"]["b"].reshape(1, _D),
      p["v"]["W"], p["v"]["b"].reshape(1, _D),
      p["o"]["W"], p["o"]["b"].reshape(1, _D),
      p["ln_g"].reshape(1, _D), p["ln_b"].reshape(1, _D))
    return out


def kernel(x, lin_x, params, edge_index, batch_vec, lin_edge_index):
    N = x.shape[0]
    E = edge_index.shape[1]
    G = 256
    L = 256

    node_x = _mm(x, params["atom_init"]["W"], params["atom_init"]["b"])
    edge_x = _mm(lin_x, params["bond_init"]["W"], params["bond_init"]["b"])

    src, dst = edge_index[0], edge_index[1]
    lsrc, ldst = lin_edge_index[0], lin_edge_index[1]
    gidx = src[lsrc]

    nca, cha = 6, 2048
    ncb, chb = 80, 4096
    perm_a, aidx_a, dst_a, cs_a = _sorted_edges(dst, nca, cha)
    src_a = jnp.pad(src[perm_a], (0, 256))
    perm_b, _, ldst_b, cs_b = _sorted_edges(ldst, ncb, chb)
    lsrc_b = jnp.pad(lsrc[perm_b], (0, 256))
    gidx_b = jnp.pad(gidx[perm_b], (0, 256))
    zeros_idx = jnp.zeros_like(src_a)
    zeros_tab = jnp.zeros((8, _D), jnp.float32)
    zeros_a = jnp.zeros((_D,), jnp.float32)

    for i in range(3):
        pa = params["atom_blocks"][i]
        pb = params["bond_blocks"][i]
        node_old = node_x
        edge_old = edge_x
        h_a = _mm(node_old, pa["h"]["W"], pa["h"]["b"])
        ea_a = _mm(edge_old, pa["We"])
        u_a, s_a = _edge_pass_sc(h_a, ea_a, src_a, dst_a, aidx_a, cs_a,
                                 pa["a"].reshape(-1), nca, cha)
        node_x = _finish(node_old, u_a[:N], _s_lanes(s_a, N), pa)
        h_b = _mm(edge_old, pb["h"]["W"], pb["h"]["b"])
        nw = _mm(node_old, pb["We"])
        u_b, s_b = _edge_pass_sc(h_b, nw, lsrc_b, ldst_b, gidx_b, cs_b,
                                 pb["a"].reshape(-1), ncb, chb)
        edge_x = _finish(edge_old, u_b[:E], _s_lanes(s_b, E), pb)

    e2n_full, _ = _edge_pass_sc(edge_x, zeros_tab, aidx_a, dst_a, zeros_idx,
                                cs_a, zeros_a, nca, cha)
    e2n = e2n_full[:N]
    edge_pooled = _mm(e2n, params["pool_edge"]["W"], params["pool_edge"]["b"],
                      act="relu")
    graph_z = _mm(jnp.concatenate([node_x, edge_pooled], axis=-1),
                  params["merge1"]["W"], params["merge1"]["b"], act="relu")

    counts = jnp.bincount(batch_vec, length=G)
    starts = jnp.concatenate([jnp.zeros((1,), counts.dtype),
                              jnp.cumsum(counts)[:-1]])
    pos = jnp.arange(N, dtype=batch_vec.dtype) - starts[batch_vec].astype(batch_vec.dtype)
    dense = jnp.zeros((G, L, _D), jnp.float32).at[batch_vec, pos].set(graph_z)
    mask = jnp.zeros((G, L), bool).at[batch_vec, pos].set(True)
    att_mask = jnp.where(mask, 0.0, -1e9).astype(jnp.float32)[:, None, :]

    seq = _ln(dense + _pos_enc(L, _D)[None], params["norm_g"], params["norm_b"])
    seq_z = _attention(seq, att_mask, params["attn"])
    seq_flat = seq_z[batch_vec, pos]

    z = _mm(jnp.concatenate([graph_z, seq_flat], axis=-1),
            params["merge2"]["W"], params["merge2"]["b"], act="relu")

    cnt = jnp.maximum(counts, 1).astype(jnp.float32)[:, None]

    def _pool(feat, pp, ff):
        g = jax.ops.segment_sum(feat, batch_vec, num_segments=G) / cnt
        hmid = _mm(g, pp["W"], pp["b"], act="relu", block=256)
        return _mm(hmid, ff["W"], ff["b"], block=256)

    z_out = _pool(z, params["pool_graph"], params["ffnn"])
    z1 = _pool(node_x, params["pool_z1"], params["ffnn_z1"])
    z2 = _pool(edge_pooled, params["pool_z2"], params["ffnn_z2"])
    return (z_out, z1, z2)


# speedup vs baseline: 9.0561x; 1.0389x over previous
import functools
import math

import jax
import jax.numpy as jnp
import numpy as np
from jax import lax
from jax.experimental import pallas as pl
from jax.experimental.pallas import tpu as pltpu
from jax.experimental.pallas import tpu_sc as plsc

_D = 128
_HEADS = 4
_DH = _D // _HEADS
_LN_EPS = 1e-5
_B = 128

_GDN = lax.GatherDimensionNumbers(offset_dims=(), collapsed_slice_dims=(0,),
                                  start_index_map=(0,))


def _shuf(v, idx):
    return lax.gather(v, idx[:, None], _GDN, (1,),
                      mode=lax.GatherScatterMode.PROMISE_IN_BOUNDS)


def _hsum(v, lanes):
    for d in (1, 2, 4, 8):
        v = v + _shuf(v, lanes ^ d)
    return v


def _edge_pass_body(h_hbm, ea_hbm, src_hbm, dst_hbm, aidx_hbm, cs_hbm, a_hbm,
                    u_hbm, s_hbm,
                    cs_v, a_v, src_v, dst_v, aidx_v, drel_v,
                    hs_v, hd_v, ea_v, msg_v, s_v,
                    acc_u, acc_s, sem1, sem2, sem3, semz, semu, sems,
                    *, n_chunks, chunk):
    cid = lax.axis_index("c")
    sid = lax.axis_index("s")
    rows = chunk // 16
    pltpu.sync_copy(cs_hbm, cs_v)
    pltpu.sync_copy(a_hbm, a_v)
    a_regs = [a_v[pl.ds(16 * k, 16)] for k in range(8)]
    zf = jnp.zeros((16,), jnp.float32)

    def _zero_rows(j, _):
        for k in range(8):
            s_v[j, pl.ds(16 * k, 16)] = zf
        return 0

    def _block(b, carry):
        seg_base, my_lo, my_hi, base8 = carry
        base = base8 + b * _B
        i1 = pltpu.async_copy(src_hbm.at[pl.ds(base, _B)], src_v, semz)
        i2 = pltpu.async_copy(dst_hbm.at[pl.ds(base, _B)], dst_v, semz)
        i3 = pltpu.async_copy(aidx_hbm.at[pl.ds(base, _B)], aidx_v, semz)
        i1.wait()
        i2.wait()
        i3.wait()
        c1 = pltpu.async_copy(h_hbm.at[src_v], hs_v, sem1)
        c2 = pltpu.async_copy(h_hbm.at[dst_v], hd_v, sem2)
        c3 = pltpu.async_copy(ea_hbm.at[aidx_v], ea_v, sem3)

        @pl.when(b > 0)
        def _():
            pltpu.make_async_copy(msg_v, acc_u.at[drel_v], semu).wait()
            pltpu.make_async_copy(s_v, acc_s.at[drel_v], sems).wait()

        for k in range(_B // 16):
            d = dst_v[pl.ds(16 * k, 16)] - seg_base
            drel_v[pl.ds(16 * k, 16)] = jnp.clip(d, 0, chunk - 1)
        c1.wait()
        c2.wait()
        c3.wait()

        def _edge(j, _):
            eid = base + j
            flag = jnp.where((eid >= my_lo) & (eid < my_hi), 1.0, 0.0)
            flag_v = lax.broadcast(flag, (16,))
            hs = [hs_v[j, pl.ds(16 * k, 16)] for k in range(8)]
            hd = [hd_v[j, pl.ds(16 * k, 16)] for k in range(8)]
            ea = [ea_v[j, pl.ds(16 * k, 16)] for k in range(8)]
            es = []
            for hh in range(4):
                k0, k1 = 2 * hh, 2 * hh + 1
                t0 = hs[k0] + hd[k0] + ea[k0]
                t1 = hs[k1] + hd[k1] + ea[k1]
                t0 = jnp.maximum(t0, 0.2 * t0)
                t1 = jnp.maximum(t1, 0.2 * t1)
                logit = _hsum(t0 * a_regs[k0] + t1 * a_regs[k1], lanes)
                es.append(jnp.exp(logit) * flag_v)
            for k in range(8):
                msg_v[j, pl.ds(16 * k, 16)] = (hs[k] + ea[k]) * es[k // 2]
                s_v[j, pl.ds(16 * k, 16)] = es[min(k, 3)]
            return 0

        lax.fori_loop(0, _B, _edge, 0)
        pltpu.async_copy(msg_v, acc_u.at[drel_v], semu, add=True)
        pltpu.async_copy(s_v, acc_s.at[drel_v], sems, add=True)
        return carry

    lanes = lax.iota(jnp.int32, 16)

    def _chunk(i, _):
        c = cid + 2 * i
        seg_base = c * chunk
        row0 = sid * rows
        lax.fori_loop(0, _B, _zero_rows, 0)
        z1 = pltpu.async_copy(s_v, acc_u.at[pl.ds(row0, rows)], semz)
        z2 = pltpu.async_copy(s_v, acc_s.at[pl.ds(row0, rows)], semz)
        z1.wait()
        z2.wait()
        plsc.subcore_barrier()
        e_lo = cs_v[pl.ds(c, 16)][0]
        e_hi = cs_v[pl.ds(c + 1, 16)][0]
        ln = e_hi - e_lo
        per = (ln + 15) // 16
        my_lo = jnp.minimum(e_lo + sid * per, e_hi)
        my_hi = jnp.minimum(my_lo + per, e_hi)
        base8 = (my_lo // 8) * 8
        nblk = (my_hi - base8 + _B - 1) // _B
        lax.fori_loop(0, nblk, _block, (seg_base, my_lo, my_hi, base8))

        @pl.when(nblk > 0)
        def _():
            pltpu.make_async_copy(msg_v, acc_u.at[drel_v], semu).wait()
            pltpu.make_async_copy(s_v, acc_s.at[drel_v], sems).wait()

        plsc.subcore_barrier()
        w1 = pltpu.async_copy(acc_u.at[pl.ds(row0, rows)],
                              u_hbm.at[pl.ds(seg_base + row0, rows)], semz)
        w2 = pltpu.async_copy(acc_s.at[pl.ds(row0, rows)],
                              s_hbm.at[pl.ds(seg_base + row0, rows)], semz)
        w1.wait()
        w2.wait()
        return 0

    lax.fori_loop(0, n_chunks // 2, _chunk, 0)


def _edge_pass_sc(h, ea_tab, src_s, dst_s, aidx_s, chunk_starts, a_flat,
                  n_chunks, chunk):
    assert n_chunks % 2 == 0 and chunk // 16 == _B
    nseg_pad = n_chunks * chunk
    cs_len = chunk_starts.shape[0]
    mesh = plsc.VectorSubcoreMesh(core_axis_name="c", subcore_axis_name="s")
    body = functools.partial(_edge_pass_body, n_chunks=n_chunks, chunk=chunk)
    fn = pl.kernel(
        body,
        out_type=[jax.ShapeDtypeStruct((nseg_pad, _D), jnp.float32),
                  jax.ShapeDtypeStruct((nseg_pad, _D), jnp.float32)],
        mesh=mesh,
        scratch_types=[
            pltpu.VMEM((cs_len,), jnp.int32),
            pltpu.VMEM((_D,), jnp.float32),
            pltpu.VMEM((_B,), jnp.int32),
            pltpu.VMEM((_B,), jnp.int32),
            pltpu.VMEM((_B,), jnp.int32),
            pltpu.VMEM((_B,), jnp.int32),
            pltpu.VMEM((_B, _D), jnp.float32),
            pltpu.VMEM((_B, _D), jnp.float32),
            pltpu.VMEM((_B, _D), jnp.float32),
            pltpu.VMEM((_B, _D), jnp.float32),
            pltpu.VMEM((_B, _D), jnp.float32),
            pltpu.VMEM_SHARED((chunk, _D), jnp.float32),
            pltpu.VMEM_SHARED((chunk, _D), jnp.float32),
            pltpu.SemaphoreType.DMA,
            pltpu.SemaphoreType.DMA,
            pltpu.SemaphoreType.DMA,
            pltpu.SemaphoreType.DMA,
            pltpu.SemaphoreType.DMA,
            pltpu.SemaphoreType.DMA,
        ],
    )
    return fn(h, ea_tab, src_s, dst_s, aidx_s, chunk_starts, a_flat)


def _mm_body(a_ref, w_ref, b_ref, o_ref, *, act):
    y = jnp.dot(a_ref[...], w_ref[...], preferred_element_type=jnp.float32)
    y = y + b_ref[...]
    if act == "relu":
        y = jnp.maximum(y, 0.0)
    o_ref[...] = y


def _mm(a, w, b=None, act=None, block=512):
    M, K = a.shape
    N = w.shape[1]
    Kp = ((K + 127) // 128) * 128
    Mp = ((M + block - 1) // block) * block
    if Kp != K:
        a = jnp.pad(a, ((0, 0), (0, Kp - K)))
        w = jnp.pad(w, ((0, Kp - K), (0, 0)))
    if Mp != M:
        a = jnp.pad(a, ((0, Mp - M), (0, 0)))
    if b is None:
        b = jnp.zeros((N,), jnp.float32)
    b2 = b.reshape(1, N)
    out = pl.pallas_call(
        functools.partial(_mm_body, act=act),
        grid=(Mp // block,),
        in_specs=[
            pl.BlockSpec((block, Kp), lambda i: (i, 0)),
            pl.BlockSpec((Kp, N), lambda i: (0, 0)),
            pl.BlockSpec((1, N), lambda i: (0, 0)),
        ],
        out_specs=pl.BlockSpec((block, N), lambda i: (i, 0)),
        out_shape=jax.ShapeDtypeStruct((Mp, N), jnp.float32),
    )(a, w, b2)
    return out[:M]


def _s_lanes(s_full, n):
    return s_full[:n].reshape(n, 8, 16)[:, :_HEADS, 0]


def _ln(x, g, b):
    m = jnp.mean(x, axis=-1, keepdims=True)
    v = jnp.var(x, axis=-1, keepdims=True)
    return (x - m) / jnp.sqrt(v + _LN_EPS) * g + b


def _sorted_edges(dst, n_chunks, chunk, pad=512):
    perm = jnp.argsort(dst)
    dst_s = dst[perm]
    bounds = jnp.arange(n_chunks + 1, dtype=jnp.int32) * chunk
    cs = jnp.searchsorted(dst_s, bounds).astype(jnp.int32)
    cs_len = ((n_chunks + 16) + 7) // 8 * 8
    cs = jnp.pad(cs, (0, cs_len - cs.shape[0]),
                 constant_values=dst.shape[0])
    dst_p = jnp.pad(dst_s, (0, pad))
    perm_p = jnp.pad(perm, (0, pad)).astype(jnp.int32)
    return perm, perm_p, dst_p, cs


def _finish_body(x_ref, u_ref, s_ref, w_ref, b_ref, g_ref, bb_ref, o_ref):
    s = s_ref[...]
    alpha = 1.0 / (s + 1e-16)
    u = u_ref[...].reshape(s.shape[0], _HEADS, _DH)
    agg = (u * alpha[..., None]).reshape(s.shape[0], _D)
    y = x_ref[...] + jnp.dot(agg, w_ref[...],
                             preferred_element_type=jnp.float32) + b_ref[...]
    m = jnp.mean(y, axis=-1, keepdims=True)
    v = jnp.mean((y - m) * (y - m), axis=-1, keepdims=True)
    o_ref[...] = (y - m) / jnp.sqrt(v + _LN_EPS) * g_ref[...] + bb_ref[...]


def _finish(x, u, s, p, block=512):
    M = x.shape[0]
    Mp = ((M + block - 1) // block) * block
    if Mp != M:
        x = jnp.pad(x, ((0, Mp - M), (0, 0)))
        u = jnp.pad(u, ((0, Mp - M), (0, 0)))
        s = jnp.pad(s, ((0, Mp - M), (0, 0)), constant_values=1.0)
    out = pl.pallas_call(
        _finish_body,
        grid=(Mp // block,),
        in_specs=[
            pl.BlockSpec((block, _D), lambda i: (i, 0)),
            pl.BlockSpec((block, _D), lambda i: (i, 0)),
            pl.BlockSpec((block, _HEADS), lambda i: (i, 0)),
            pl.BlockSpec((_D, _D), lambda i: (0, 0)),
            pl.BlockSpec((1, _D), lambda i: (0, 0)),
            pl.BlockSpec((1, _D), lambda i: (0, 0)),
            pl.BlockSpec((1, _D), lambda i: (0, 0)),
        ],
        out_specs=pl.BlockSpec((block, _D), lambda i: (i, 0)),
        out_shape=jax.ShapeDtypeStruct((Mp, _D), jnp.float32),
    )(x, u, s, p["o"]["W"], p["o"]["b"].reshape(1, _D),
      p["ln_g"].reshape(1, _D), p["ln_b"].reshape(1, _D))
    return out[:M]


def _pos_enc(L, d):
    pos = np.arange(L)[:, None].astype(np.float32)
    i = np.arange(d)[None, :].astype(np.float32)
    ang = pos / np.power(10000.0, (2.0 * np.floor(i / 2.0)) / d)
    pe = np.zeros((L, d), np.float32)
    pe[:, 0::2] = np.sin(ang[:, 0::2])
    pe[:, 1::2] = np.cos(ang[:, 1::2])
    return jnp.asarray(pe)


def _attn_body(x_ref, mask_ref, wq_ref, bq_ref, wk_ref, bk_ref, wv_ref,
               bv_ref, wo_ref, bo_ref, g_ref, bb_ref, o_ref):
    x = x_ref[0]
    q = jnp.dot(x, wq_ref[...], preferred_element_type=jnp.float32) + bq_ref[...]
    k = jnp.dot(x, wk_ref[...], preferred_element_type=jnp.float32) + bk_ref[...]
    v = jnp.dot(x, wv_ref[...], preferred_element_type=jnp.float32) + bv_ref[...]
    mask = mask_ref[0]
    outs = []
    scale = 1.0 / math.sqrt(_DH)
    for h in range(_HEADS):
        qh = q[:, h * _DH:(h + 1) * _DH]
        kh = k[:, h * _DH:(h + 1) * _DH]
        vh = v[:, h * _DH:(h + 1) * _DH]
        sc = jax.lax.dot_general(qh, kh, (((1,), (1,)), ((), ())),
                                 preferred_element_type=jnp.float32)
        sc = sc * scale + mask
        sc = sc - jnp.max(sc, axis=-1, keepdims=True)
        p = jnp.exp(sc)
        p = p / jnp.sum(p, axis=-1, keepdims=True)
        outs.append(jnp.dot(p, vh, preferred_element_type=jnp.float32))
    o = jnp.concatenate(outs, axis=-1)
    y = x + jnp.dot(o, wo_ref[...], preferred_element_type=jnp.float32) + bo_ref[...]
    m = jnp.mean(y, axis=-1, keepdims=True)
    var = jnp.mean((y - m) * (y - m), axis=-1, keepdims=True)
    o_ref[0] = (y - m) / jnp.sqrt(var + _LN_EPS) * g_ref[...] + bb_ref[...]


def _attention(x, att_mask, p):
    G, L, _ = x.shape
    out = pl.pallas_call(
        _attn_body,
        grid=(G,),
        in_specs=[
            pl.BlockSpec((1, L, _D), lambda i: (i, 0, 0)),
            pl.BlockSpec((1, 1, L), lambda i: (i, 0, 0)),
            pl.BlockSpec((_D, _D), lambda i: (0, 0)),
            pl.BlockSpec((1, _D), lambda i: (0, 0)),
            pl.BlockSpec((_D, _D), lambda i: (0, 0)),
            pl.BlockSpec((1, _D), lambda i: (0, 0)),
            pl.BlockSpec((_D, _D), lambda i: (0, 0)),
            pl.BlockSpec((1, _D), lambda i: (0, 0)),
            pl.BlockSpec((_D, _D), lambda i: (0, 0)),
            pl.BlockSpec((1, _D), lambda i: (0, 0)),
            pl.BlockSpec((1, _D), lambda i: (0, 0)),
            pl.BlockSpec((1, _D), lambda i: (0, 0)),
        ],
        out_specs=pl.BlockSpec((1, L, _D), lambda i: (i, 0, 0)),
        out_shape=jax.ShapeDtypeStruct((G, L, _D), jnp.float32),
    )(x, att_mask,
      p["q"]["W"], p["q"]["b"].reshape(1, _D),
      p["k"]["W"], p["k"]["b"].reshape(1, _D),
      p["v"]["W"], p["v"]["b"].reshape(1, _D),
      p["o"]["W"], p["o"]["b"].reshape(1, _D),
      p["ln_g"].reshape(1, _D), p["ln_b"].reshape(1, _D))
    return out


def kernel(x, lin_x, params, edge_index, batch_vec, lin_edge_index):
    N = x.shape[0]
    E = edge_index.shape[1]
    G = 256
    L = 256

    node_x = _mm(x, params["atom_init"]["W"], params["atom_init"]["b"])
    edge_x = _mm(lin_x, params["bond_init"]["W"], params["bond_init"]["b"])

    src, dst = edge_index[0], edge_index[1]
    lsrc, ldst = lin_edge_index[0], lin_edge_index[1]
    gidx = src[lsrc]

    nca, cha = 6, 2048
    ncb, chb = 160, 2048
    perm_a, aidx_a, dst_a, cs_a = _sorted_edges(dst, nca, cha, pad=512)
    src_a = jnp.pad(src[perm_a], (0, 512))
    perm_b, _, ldst_b, cs_b = _sorted_edges(ldst, ncb, chb, pad=512)
    lsrc_b = jnp.pad(lsrc[perm_b], (0, 512))
    gidx_b = jnp.pad(gidx[perm_b], (0, 512))
    zeros_idx = jnp.zeros_like(src_a)
    zeros_tab = jnp.zeros((8, _D), jnp.float32)
    zeros_a = jnp.zeros((_D,), jnp.float32)

    for i in range(3):
        pa = params["atom_blocks"][i]
        pb = params["bond_blocks"][i]
        node_old = node_x
        edge_old = edge_x
        h_a = _mm(node_old, pa["h"]["W"], pa["h"]["b"])
        ea_a = _mm(edge_old, pa["We"])
        u_a, s_a = _edge_pass_sc(h_a, ea_a, src_a, dst_a, aidx_a, cs_a,
                                 pa["a"].reshape(-1), nca, cha)
        node_x = _finish(node_old, u_a[:N], _s_lanes(s_a, N), pa)
        h_b = _mm(edge_old, pb["h"]["W"], pb["h"]["b"])
        nw = _mm(node_old, pb["We"])
        u_b, s_b = _edge_pass_sc(h_b, nw, lsrc_b, ldst_b, gidx_b, cs_b,
                                 pb["a"].reshape(-1), ncb, chb)
        edge_x = _finish(edge_old, u_b[:E], _s_lanes(s_b, E), pb)

    e2n_full, _ = _edge_pass_sc(edge_x, zeros_tab, aidx_a, dst_a, zeros_idx,
                                cs_a, zeros_a, nca, cha)
    e2n = e2n_full[:N]
    edge_pooled = _mm(e2n, params["pool_edge"]["W"], params["pool_edge"]["b"],
                      act="relu")
    graph_z = _mm(jnp.concatenate([node_x, edge_pooled], axis=-1),
                  params["merge1"]["W"], params["merge1"]["b"], act="relu")

    counts = jnp.bincount(batch_vec, length=G)
    starts = jnp.concatenate([jnp.zeros((1,), counts.dtype),
                              jnp.cumsum(counts)[:-1]])
    pos = jnp.arange(N, dtype=batch_vec.dtype) - starts[batch_vec].astype(batch_vec.dtype)
    dense = jnp.zeros((G, L, _D), jnp.float32).at[batch_vec, pos].set(graph_z)
    mask = jnp.zeros((G, L), bool).at[batch_vec, pos].set(True)
    att_mask = jnp.where(mask, 0.0, -1e9).astype(jnp.float32)[:, None, :]

    seq = _ln(dense + _pos_enc(L, _D)[None], params["norm_g"], params["norm_b"])
    seq_z = _attention(seq, att_mask, params["attn"])
    seq_flat = seq_z[batch_vec, pos]

    z = _mm(jnp.concatenate([graph_z, seq_flat], axis=-1),
            params["merge2"]["W"], params["merge2"]["b"], act="relu")

    cnt = jnp.maximum(counts, 1).astype(jnp.float32)[:, None]

    def _pool(feat, pp, ff):
        g = jax.ops.segment_sum(feat, batch_vec, num_segments=G) / cnt
        hmid = _mm(g, pp["W"], pp["b"], act="relu", block=256)
        return _mm(hmid, ff["W"], ff["b"], block=256)

    z_out = _pool(z, params["pool_graph"], params["ffnn"])
    z1 = _pool(node_x, params["pool_z1"], params["ffnn_z1"])
    z2 = _pool(edge_pooled, params["pool_z2"], params["ffnn_z2"])
    return (z_out, z1, z2)
